# Initial kernel scaffold; baseline (speedup 1.0000x reference)
import jax
import jax.numpy as jnp
from jax.experimental import pallas as pl


def kernel(x, edge_index, W1_self, W1_neigh, b1, W2_self, W2_neigh, b2, Wc1, bc1, Wc2, bc2):
    raise NotImplementedError("write your pallas kernel here")


# speedup vs baseline: 3.2723x; 3.2723x over previous
import functools

import jax
import jax.numpy as jnp
from jax import lax
from jax.experimental import pallas as pl
from jax.experimental.pallas import tpu as pltpu
from jax.experimental.pallas import tpu_sc as plsc

_N = 10000
_E = 320000
_D = 128
_CHUNK = 128
_NC = 2
_NS = 16
_NW = _NC * _NS
_EROWS = -(-_E // _CHUNK)
_EROWS_PAD = -(-_EROWS // (_NW * 8)) * _NW * 8
_RPW = _EROWS_PAD // _NW
_ACC_ROWS = 10240
_ZROWS = _ACC_ROWS // _NS
_WROWS = 624
_WLAST = _N - 15 * _WROWS


def _make_sc_agg(with_count: bool):
  mesh = plsc.VectorSubcoreMesh(core_axis_name="c", subcore_axis_name="s")
  sums_type = jax.ShapeDtypeStruct((_NC, _N, _D), jnp.float32)
  out_type = sums_type
  scratch = [
      pltpu.VMEM((8, _CHUNK), jnp.int32),
      pltpu.VMEM((8, _CHUNK), jnp.int32),
      pltpu.VMEM((_CHUNK, _D), jnp.float32),
      pltpu.VMEM_SHARED((_ACC_ROWS, _D), jnp.float32),
  ]
  if with_count:
    out_type = [sums_type, jax.ShapeDtypeStruct((_NC, _N, 16), jnp.float32)]
    scratch += [
        pltpu.VMEM((_CHUNK, 16), jnp.float32),
        pltpu.VMEM_SHARED((_ACC_ROWS, 16), jnp.float32),
    ]

  def body(*refs):
    if with_count:
      (h_hbm, srcr, dstr, z128, z16, ones_h,
       sums_hbm, cnt_hbm, idx_s, idx_d, rows_v, acc, ones_v, cacc) = refs
    else:
      (h_hbm, srcr, dstr, z128,
       sums_hbm, idx_s, idx_d, rows_v, acc) = refs
    cid = lax.axis_index("c")
    sid = lax.axis_index("s")
    wid = cid * _NS + sid

    z0 = sid * _ZROWS
    pltpu.sync_copy(z128.at[pl.ds(0, _ZROWS)], acc.at[pl.ds(z0, _ZROWS)])
    if with_count:
      pltpu.sync_copy(z16.at[pl.ds(0, _ZROWS)], cacc.at[pl.ds(z0, _ZROWS)])
      pltpu.sync_copy(ones_h, ones_v)

    r0 = wid * _RPW
    plsc.subcore_barrier()

    @pl.loop(0, _RPW // 8)
    def _(c):
      pltpu.sync_copy(srcr.at[pl.ds(r0 + c * 8, 8)], idx_s)
      pltpu.sync_copy(dstr.at[pl.ds(r0 + c * 8, 8)], idx_d)

      @pl.loop(0, 8)
      def _(j):
        pltpu.sync_copy(h_hbm.at[idx_s.at[j]], rows_v)
        pltpu.sync_copy(rows_v, acc.at[idx_d.at[j]], add=True)
        if with_count:
          pltpu.sync_copy(ones_v, cacc.at[idx_d.at[j]], add=True)

    plsc.subcore_barrier()

    w0 = sid * _WROWS

    @pl.when(sid < _NS - 1)
    def _():
      pltpu.sync_copy(acc.at[pl.ds(w0, _WROWS)],
                      sums_hbm.at[cid].at[pl.ds(w0, _WROWS)])
      if with_count:
        pltpu.sync_copy(cacc.at[pl.ds(w0, _WROWS)],
                        cnt_hbm.at[cid].at[pl.ds(w0, _WROWS)])

    @pl.when(sid == _NS - 1)
    def _():
      w1 = (_NS - 1) * _WROWS
      pltpu.sync_copy(acc.at[pl.ds(w1, _WLAST)],
                      sums_hbm.at[cid].at[pl.ds(w1, _WLAST)])
      if with_count:
        pltpu.sync_copy(cacc.at[pl.ds(w1, _WLAST)],
                        cnt_hbm.at[cid].at[pl.ds(w1, _WLAST)])

  return pl.kernel(
      body, out_type=out_type, mesh=mesh, scratch_types=scratch,
      compiler_params=pltpu.CompilerParams(use_tc_tiling_on_sc=False))


_sc_agg_count = _make_sc_agg(True)
_sc_agg = _make_sc_agg(False)


_R = 1000


def _dense1_body(x_r, s_r, c_r, ws_r, wn_r, b_r, o_r):
  cnt = c_r[0, :, 0:1] + c_r[1, :, 0:1]
  inv = 1.0 / jnp.maximum(cnt, 1.0)
  mean = (s_r[0] + s_r[1]) * inv
  h = (jnp.dot(x_r[...], ws_r[...], preferred_element_type=jnp.float32)
       + jnp.dot(mean, wn_r[...], preferred_element_type=jnp.float32)
       + b_r[...])
  o_r[...] = jnp.maximum(h, 0.0)


def _dense2_body(h_r, s_r, c_r, ws_r, wn_r, b_r, wc1_r, bc1_r, wc2_r, bc2_r, o_r):
  cnt = c_r[0, :, 0:1] + c_r[1, :, 0:1]
  inv = 1.0 / jnp.maximum(cnt, 1.0)
  mean = (s_r[0] + s_r[1]) * inv
  h2 = (jnp.dot(h_r[...], ws_r[...], preferred_element_type=jnp.float32)
        + jnp.dot(mean, wn_r[...], preferred_element_type=jnp.float32)
        + b_r[...])
  h2 = jnp.maximum(h2, 0.0)
  hid = jnp.maximum(
      jnp.dot(h2, wc1_r[...], preferred_element_type=jnp.float32) + bc1_r[...],
      0.0)
  o_r[...] = (jnp.dot(hid, wc2_r[...], preferred_element_type=jnp.float32)
              + bc2_r[...])


def _row_spec(w):
  return pl.BlockSpec((_R, w), lambda i: (i, 0))


def _part_spec(w):
  return pl.BlockSpec((_NC, _R, w), lambda i: (0, i, 0))


def _full_spec(h, w):
  return pl.BlockSpec((h, w), lambda i: (0, 0))


def _dense1(x, sums, cnts, Ws, Wn, b):
  return pl.pallas_call(
      _dense1_body,
      grid=(_N // _R,),
      in_specs=[
          _row_spec(_D), _part_spec(_D), _part_spec(16),
          _full_spec(_D, _D), _full_spec(_D, _D), _full_spec(1, _D),
      ],
      out_specs=_row_spec(_D),
      out_shape=jax.ShapeDtypeStruct((_N, _D), jnp.float32),
  )(x, sums, cnts, Ws, Wn, b.reshape(1, _D))


def _dense2(h, sums, cnts, Ws, Wn, b, Wc1, bc1, Wc2, bc2):
  n_cls = Wc2.shape[1]
  cls_hid = Wc1.shape[1]
  return pl.pallas_call(
      _dense2_body,
      grid=(_N // _R,),
      in_specs=[
          _row_spec(_D), _part_spec(_D), _part_spec(16),
          _full_spec(_D, _D), _full_spec(_D, _D), _full_spec(1, _D),
          _full_spec(_D, cls_hid), _full_spec(1, cls_hid),
          _full_spec(cls_hid, n_cls), _full_spec(1, n_cls),
      ],
      out_specs=_row_spec(n_cls),
      out_shape=jax.ShapeDtypeStruct((_N, n_cls), jnp.float32),
  )(h, sums, cnts, Ws, Wn, b.reshape(1, _D), Wc1, bc1.reshape(1, cls_hid),
    Wc2, bc2.reshape(1, n_cls))


def kernel(x, edge_index, W1_self, W1_neigh, b1, W2_self, W2_neigh, b2,
           Wc1, bc1, Wc2, bc2):
  pad = _EROWS_PAD * _CHUNK - _E
  src = jnp.concatenate([edge_index[0], jnp.zeros((pad,), jnp.int32)])
  dst = jnp.concatenate([edge_index[1], jnp.full((pad,), _N, jnp.int32)])
  src = src.reshape(_EROWS_PAD, _CHUNK)
  dst = dst.reshape(_EROWS_PAD, _CHUNK)

  z128 = jnp.zeros((_ZROWS, _D), jnp.float32)
  z16 = jnp.zeros((_ZROWS, 16), jnp.float32)
  ones16 = jnp.ones((_CHUNK, 16), jnp.float32)

  sums1, cnts = _sc_agg_count(x, src, dst, z128, z16, ones16)
  h1 = _dense1(x, sums1, cnts, W1_self, W1_neigh, b1)
  sums2 = _sc_agg(h1, src, dst, z128)
  return _dense2(h1, sums2, cnts, W2_self, W2_neigh, b2, Wc1, bc1, Wc2, bc2)
